# baseline (speedup 1.0000x reference)
import functools

import jax
import jax.numpy as jnp
from jax import lax
from jax.experimental import pallas as pl
from jax.experimental.pallas import tpu as pltpu
from jax.experimental.pallas import tpu_sc as plsc

H = 128
OUT = 64
T = 40
NPG = 1250
EPG = 8000
N = T * NPG
E = T * EPG
SENT = 128
N_REL = 256
MT = T * N_REL

NC = 2
NS = 16
NW = NC * NS
LANES = 16

CE = 320
NB = CE // LANES
NCH = EPG // CE
NPH = 625
NPHP = 640

_DOT = functools.partial(jnp.dot, preferred_element_type=jnp.float32,
                         precision=lax.Precision.HIGHEST)


def _sc_mesh():
    return plsc.VectorSubcoreMesh(core_axis_name="c", subcore_axis_name="s",
                                  num_cores=NC, num_subcores=NS)


def _wid():
    return lax.axis_index("c") * NS + lax.axis_index("s")


_SC_PARAMS = pltpu.CompilerParams(needs_layout_passes=False)


def _sc_gather_rows(table, idx, chunk):
    V, D = table.shape
    B = idx.shape[0]
    per_w = B // NW
    n_chunks = per_w // chunk
    n_outer = (n_chunks + 1) // 2

    @functools.partial(
        pl.kernel,
        compiler_params=_SC_PARAMS,
        out_type=jax.ShapeDtypeStruct((B, D), jnp.float32),
        mesh=_sc_mesh(),
        scratch_types=[
            pltpu.VMEM((per_w,), jnp.int32),
            pltpu.VMEM((chunk, D), jnp.float32),
            pltpu.VMEM((chunk, D), jnp.float32),
            pltpu.SemaphoreType.DMA,
            pltpu.SemaphoreType.DMA,
        ],
    )
    def k(table_hbm, idx_hbm, out_hbm, idx_v, rows0, rows1, sem0, sem1):
        base = _wid() * per_w
        pltpu.sync_copy(idx_hbm.at[pl.ds(pl.multiple_of(base, 8), per_w)],
                        idx_v)
        rows = (rows0, rows1)
        sems = (sem0, sem1)

        def start(i, b):
            pltpu.async_copy(
                table_hbm.at[idx_v.at[pl.ds(i * chunk, chunk)]],
                rows[b], sems[b])

        def finish(i, b):
            pltpu.make_async_copy(
                table_hbm.at[idx_v.at[pl.ds(0, chunk)]],
                rows[b], sems[b]).wait()
            off = pl.multiple_of(base + i * chunk, 8)
            pltpu.sync_copy(rows[b], out_hbm.at[pl.ds(off, chunk)])

        start(0, 0)

        def outer(o, carry):
            i0 = 2 * o

            @pl.when(i0 + 1 < n_chunks)
            def _():
                start(i0 + 1, 1)
            finish(i0, 0)

            @pl.when(i0 + 2 < n_chunks)
            def _():
                start(i0 + 2, 0)

            @pl.when(i0 + 1 < n_chunks)
            def _():
                finish(i0 + 1, 1)
            return carry

        lax.fori_loop(0, n_outer, outer, 0)

    return k(table, idx)


def _sc_scatter_graph(msg, dst, n_half, want_deg):
    FP = 64
    shift = n_half - 1
    n_units = T * 2 * n_half

    out_types = [jax.ShapeDtypeStruct((n_half, T, 2, NPHP * FP), jnp.float32)]
    if want_deg:
        out_types.append(jax.ShapeDtypeStruct((T * 2 * NPHP,), jnp.float32))

    @functools.partial(
        pl.kernel,
        compiler_params=_SC_PARAMS,
        out_type=tuple(out_types),
        mesh=_sc_mesh(),
        scratch_types=[
            pltpu.VMEM((CE,), jnp.int32),
            pltpu.VMEM((CE, FP), jnp.float32),
            pltpu.VMEM((NPHP * FP,), jnp.float32),
            pltpu.VMEM((NPHP,), jnp.float32),
        ],
    )
    def k(msg_hbm, dst_hbm, *refs):
        if want_deg:
            agg_hbm, deg_hbm, dst_v, msg_v, acc, dacc = refs
        else:
            (agg_hbm, dst_v, msg_v, acc, dacc) = refs
            deg_hbm = None
        w = _wid()
        n_my = ((n_units - 1 - w) >> 5) + 1
        iota = lax.iota(jnp.int32, LANES)
        ones = jnp.ones((LANES,), jnp.float32)
        zeros = jnp.zeros((LANES,), jnp.float32)

        def unit_body(ui, carry):
            u = w + ui * NW
            g = u >> (shift + 1)
            nh = (u >> shift) & 1
            fp = u & shift

            def zrow(r, c):
                for jj in range(FP // LANES):
                    acc[pl.ds((r * (FP // LANES) + jj) * LANES, LANES)] = zeros
                return c
            lax.fori_loop(0, NPHP, zrow, 0)
            if want_deg:
                def zdeg(r, c):
                    dacc[pl.ds(r * LANES, LANES)] = zeros
                    return c
                lax.fori_loop(0, NPHP // LANES, zdeg, 0)

            nhbase = g * NPG + nh * NPH

            def chunk_body(ci, c2):
                off = pl.multiple_of(g * EPG + ci * CE, 8)
                pltpu.sync_copy(dst_hbm.at[pl.ds(off, CE)], dst_v)
                pltpu.sync_copy(msg_hbm.at[fp, pl.ds(off, CE)], msg_v)

                def blk_body(b, c3):
                    d16 = dst_v[pl.ds(b * LANES, LANES)] - nhbase
                    valid = (d16 >= 0) & (d16 < NPH)
                    d16c = jnp.where(valid, d16, NPH)
                    dflat = d16c * FP
                    if want_deg:
                        plsc.addupdate_scatter(dacc, [d16c], ones, mask=valid)
                    for l in range(LANES):
                        dl = dflat.at[jnp.full((LANES,), l, jnp.int32)].get(
                            mode="promise_in_bounds") + iota
                        for jj in range(FP // LANES):
                            v = msg_v[b * LANES + l, pl.ds(jj * LANES, LANES)]
                            plsc.addupdate_scatter(acc, [dl + jj * LANES], v)
                    return c3

                lax.fori_loop(0, NB, blk_body, 0)
                return c2

            lax.fori_loop(0, NCH, chunk_body, 0)

            pltpu.sync_copy(acc, agg_hbm.at[fp, g, nh])
            if want_deg:
                doff = pl.multiple_of((g * 2 + nh) * NPHP, 8)
                pltpu.sync_copy(dacc, deg_hbm.at[pl.ds(doff, NPHP)])
            return carry

        lax.fori_loop(0, n_my, unit_body, 0)

    outs = k(msg, dst)
    agg = outs[0].reshape(n_half, T, 2, NPHP, FP)
    return (agg, outs[1]) if want_deg else (agg,)


def _sc_type_mask(types):
    @functools.partial(
        pl.kernel,
        compiler_params=_SC_PARAMS,
        out_type=jax.ShapeDtypeStruct((NW, 2 * T, 128), jnp.float32),
        mesh=_sc_mesh(),
        scratch_types=[
            pltpu.VMEM((CE,), jnp.int32),
            pltpu.VMEM((2 * T, 128), jnp.float32),
        ],
    )
    def k(types_hbm, out_hbm, tv, macc):
        w = _wid()
        n_my = ((T - 1 - w) >> 5) + 1
        ones = jnp.ones((LANES,), jnp.float32)
        zeros = jnp.zeros((LANES,), jnp.float32)

        def zrow(r, c):
            for jj in range(128 // LANES):
                macc[r, pl.ds(jj * LANES, LANES)] = zeros
            return c
        lax.fori_loop(0, 2 * T, zrow, 0)

        def unit_body(ui, carry):
            g = w + ui * NW
            goff = g * N_REL

            def chunk_body(ci, c):
                off = pl.multiple_of(g * EPG + ci * CE, 8)
                pltpu.sync_copy(types_hbm.at[pl.ds(off, CE)], tv)

                def blk(b, c2):
                    t16 = tv[pl.ds(b * LANES, LANES)] + goff
                    plsc.addupdate_scatter(
                        macc, [t16 >> 7, t16 & 127], ones)
                    return c2

                lax.fori_loop(0, NB, blk, 0)
                return c

            lax.fori_loop(0, NCH, chunk_body, 0)
            return carry

        lax.fori_loop(0, n_my, unit_body, 0)
        pltpu.sync_copy(macc, out_hbm.at[w])

    return k(types)


def _tc_precompute(ent_embeds, ent_memory, rel_embeds, rel_memory,
                   Wn, bn, Wr, br, Wre1, Wre2):
    def body(ee, em, re, rm, wn, bn_, wr, br_, w1, w2,
             he_o, re0_o, re1_o, re2_o):
        he_o[...] = _DOT(ee[...], wn[0:2 * H, :]) + \
            _DOT(em[...], wn[2 * H:3 * H, :]) + bn_[...]
        re0 = _DOT(re[...], wr[0:2 * H, :]) + \
            _DOT(rm[...], wr[2 * H:3 * H, :]) + br_[...]
        re0_o[...] = re0
        re1 = jnp.maximum(_DOT(re0, w1[...]), 0.0)
        re1_o[...] = re1
        re2_o[...] = jnp.maximum(_DOT(re1, w2[...]), 0.0)

    n_ent = ent_embeds.shape[0]
    BN = 2000
    return pl.pallas_call(
        body,
        grid=(n_ent // BN,),
        in_specs=[
            pl.BlockSpec((BN, 2 * H), lambda i: (i, 0)),
            pl.BlockSpec((BN, H), lambda i: (i, 0)),
            pl.BlockSpec((N_REL, 2 * H), lambda i: (0, 0)),
            pl.BlockSpec((N_REL, H), lambda i: (0, 0)),
            pl.BlockSpec((3 * H, H), lambda i: (0, 0)),
            pl.BlockSpec((1, H), lambda i: (0, 0)),
            pl.BlockSpec((3 * H, H), lambda i: (0, 0)),
            pl.BlockSpec((1, H), lambda i: (0, 0)),
            pl.BlockSpec((H, OUT), lambda i: (0, 0)),
            pl.BlockSpec((OUT, H), lambda i: (0, 0)),
        ],
        out_specs=(
            pl.BlockSpec((BN, H), lambda i: (i, 0)),
            pl.BlockSpec((N_REL, H), lambda i: (0, 0)),
            pl.BlockSpec((N_REL, OUT), lambda i: (0, 0)),
            pl.BlockSpec((N_REL, H), lambda i: (0, 0)),
        ),
        out_shape=(
            jax.ShapeDtypeStruct((n_ent, H), jnp.float32),
            jax.ShapeDtypeStruct((N_REL, H), jnp.float32),
            jax.ShapeDtypeStruct((N_REL, OUT), jnp.float32),
            jax.ShapeDtypeStruct((N_REL, H), jnp.float32),
        ),
    )(ent_embeds, ent_memory, rel_embeds, rel_memory,
      Wn, bn.reshape(1, H), Wr, br.reshape(1, H), Wre1, Wre2)


def _tc_msg(hs, types_col, re_tab, text, Wm, Wt, d_use, n_half):
    BE = 3200
    FP = 64
    wm3 = jnp.moveaxis(Wm.reshape(d_use, n_half, FP), 1, 0)
    wt3 = jnp.moveaxis(Wt.reshape(SENT, n_half, FP), 1, 0)

    def body(hs_r, t_r, re_r, tx_r, wm_r, wt_r, out_r):
        oh = (lax.broadcasted_iota(jnp.int32, (BE, N_REL), 1) ==
              t_r[...]).astype(jnp.float32)
        es = _DOT(oh, re_r[...])
        prod = hs_r[...][:, :d_use] * es
        outs = [_DOT(prod, wm_r[...][f]) + _DOT(tx_r[...], wt_r[...][f])
                for f in range(n_half)]
        out_r[...] = jnp.stack(outs, axis=0)

    return pl.pallas_call(
        body,
        grid=(E // BE,),
        in_specs=[
            pl.BlockSpec((BE, H), lambda i: (i, 0)),
            pl.BlockSpec((BE, 1), lambda i: (i, 0)),
            pl.BlockSpec((N_REL, d_use), lambda i: (0, 0)),
            pl.BlockSpec((BE, SENT), lambda i: (i, 0)),
            pl.BlockSpec((n_half, d_use, FP), lambda i: (0, 0, 0)),
            pl.BlockSpec((n_half, SENT, FP), lambda i: (0, 0, 0)),
        ],
        out_specs=pl.BlockSpec((n_half, BE, FP), lambda i: (0, i, 0)),
        out_shape=jax.ShapeDtypeStruct((n_half, E, FP), jnp.float32),
    )(hs, types_col, re_tab, text, wm3, wt3)


def _assemble_agg(a, n_half):
    halves = []
    for f in range(n_half):
        rows = jnp.concatenate([a[f, 0, 0, :NPH, :], a[f, 0, 1, :NPH, :]],
                               axis=0)
        halves.append(rows)
    return jnp.concatenate(halves, axis=-1) if n_half > 1 else halves[0]


def _assemble_deg(dr):
    d = jnp.concatenate([dr[0, 0, :NPH], dr[0, 0, NPHP:NPHP + NPH]], axis=0)
    return jnp.maximum(d, 1.0)[:, None]


def _tc_update(agg5, deg3, h3d, Wl, d_in, d_out, n_half, pad_to=None):
    width = pad_to or d_out

    def body(agg_r, deg_r, h_r, wl_r, out_r):
        agg = _assemble_agg(agg_r[...], n_half)
        d = _assemble_deg(deg_r[...])
        h2 = jnp.maximum(agg / d + _DOT(h_r[...][0], wl_r[...]), 0.0)
        if width > d_out:
            h2 = jnp.concatenate(
                [h2, jnp.zeros((NPG, width - d_out), jnp.float32)], axis=-1)
        out_r[...] = h2[None]

    return pl.pallas_call(
        body,
        grid=(T,),
        in_specs=[
            pl.BlockSpec((n_half, 1, 2, NPHP, 64), lambda i: (0, i, 0, 0, 0)),
            pl.BlockSpec((1, 1, 2 * NPHP), lambda i: (i, 0, 0)),
            pl.BlockSpec((1, NPG, d_in), lambda i: (i, 0, 0)),
            pl.BlockSpec((d_in, d_out), lambda i: (0, 0)),
        ],
        out_specs=pl.BlockSpec((1, NPG, width), lambda i: (i, 0, 0)),
        out_shape=jax.ShapeDtypeStruct((T, NPG, width), jnp.float32),
    )(agg5, deg3, h3d, Wl)


def _tc_pool(agg5, deg3, h2_3d, Wl2, mask2d, re2):
    def body(agg_r, deg_r, h2_r, wl_r, mask_r, re2_r, g_o):
        i = pl.program_id(0)
        agg = _assemble_agg(agg_r[...], 2)
        d = _assemble_deg(deg_r[...])
        h3 = jnp.maximum(agg / d + _DOT(h2_r[...][0][:, :OUT], wl_r[...]),
                         0.0)
        gn = jnp.max(h3, axis=0, keepdims=True)
        m = jnp.sum(mask_r[:, pl.ds(i * N_REL, N_REL)], axis=0)
        ge = jnp.max(jnp.where(m[:, None] > 0.0, re2_r[...], -3.4e38),
                     axis=0, keepdims=True)
        g_o[...] = jnp.concatenate([gn, ge], axis=1)[None]

    return pl.pallas_call(
        body,
        grid=(T,),
        in_specs=[
            pl.BlockSpec((2, 1, 2, NPHP, 64), lambda i: (0, i, 0, 0, 0)),
            pl.BlockSpec((1, 1, 2 * NPHP), lambda i: (i, 0, 0)),
            pl.BlockSpec((1, NPG, H), lambda i: (i, 0, 0)),
            pl.BlockSpec((OUT, H), lambda i: (0, 0)),
            pl.BlockSpec((NW, MT), lambda i: (0, 0)),
            pl.BlockSpec((N_REL, H), lambda i: (0, 0)),
        ],
        out_specs=pl.BlockSpec((1, 1, 2 * H), lambda i: (i, 0, 0)),
        out_shape=jax.ShapeDtypeStruct((T, 1, 2 * H), jnp.float32),
    )(agg5, deg3, h2_3d, Wl2, mask2d, re2)


def _tc_final(g, time_idx_col, n_q):
    def body(g_r, ti_r, out_r):
        oh = (lax.broadcasted_iota(jnp.int32, (n_q, T), 1) ==
              ti_r[...]).astype(jnp.float32)
        out_r[...] = _DOT(oh, g_r[...])

    return pl.pallas_call(
        body,
        out_shape=jax.ShapeDtypeStruct((n_q, 2 * H), jnp.float32),
    )(g, time_idx_col)


def kernel(node_ids, edge_types, edge_index, node_graph_ids, edge_graph_ids,
           time_idx, ent_embeds, ent_memory, rel_embeds, rel_memory, text_emb,
           Wn, bn, Wr, br, Wm1, Wl1, Wre1, Wt1, Wm2, Wl2, Wre2, Wt2):
    src = edge_index[0].astype(jnp.int32)
    dst = edge_index[1].astype(jnp.int32)
    types = edge_types.astype(jnp.int32)
    text = text_emb.astype(jnp.float32)

    he, re0, re1, re2 = _tc_precompute(
        ent_embeds.astype(jnp.float32), ent_memory.astype(jnp.float32),
        rel_embeds.astype(jnp.float32), rel_memory.astype(jnp.float32),
        Wn, bn, Wr, br, Wre1, Wre2)

    NPAD = 50176
    nid_pad = jnp.pad(node_ids.astype(jnp.int32), (0, NPAD - N))
    h1 = _sc_gather_rows(he, nid_pad, 224)[:N]

    hs1 = _sc_gather_rows(h1, src, 200)
    mask_part = _sc_type_mask(types)
    types_col = types.reshape(E, 1)

    msg1 = _tc_msg(hs1, types_col, re0, text, Wm1, Wt1, H, 1)
    agg1, deg1d = _sc_scatter_graph(msg1, dst, 1, True)
    deg3 = deg1d.reshape(T, 1, 2 * NPHP)
    h2_3d = _tc_update(agg1, deg3, h1.reshape(T, NPG, H), Wl1, H, OUT, 1,
                       pad_to=H)
    h2 = h2_3d.reshape(N, H)

    hs2 = _sc_gather_rows(h2, src, 200)
    msg2 = _tc_msg(hs2, types_col, re1, text, Wm2, Wt2, OUT, 2)
    agg2 = _sc_scatter_graph(msg2, dst, 2, False)[0]

    g = _tc_pool(agg2, deg3, h2_3d, Wl2,
                 mask_part.reshape(NW, MT), re2).reshape(T, 2 * H)
    bq, sq = time_idx.shape
    ti_col = time_idx.astype(jnp.int32).reshape(bq * sq, 1)
    out = _tc_final(g, ti_col, bq * sq)
    return out.reshape(bq, sq, 2 * H)


# speedup vs baseline: 1.1423x; 1.1423x over previous
import functools

import jax
import jax.numpy as jnp
from jax import lax
from jax.experimental import pallas as pl
from jax.experimental.pallas import tpu as pltpu
from jax.experimental.pallas import tpu_sc as plsc

H = 128
OUT = 64
T = 40
NPG = 1250
EPG = 8000
N = T * NPG
E = T * EPG
SENT = 128
N_REL = 256
MT = T * N_REL

NC = 2
NS = 16
NW = NC * NS
LANES = 16

CE = 160
NB = CE // LANES
NCH = EPG // CE
NPH = 625
NPHP = 640

_DOT = functools.partial(jnp.dot, preferred_element_type=jnp.float32,
                         precision=lax.Precision.HIGHEST)


def _sc_mesh():
    return plsc.VectorSubcoreMesh(core_axis_name="c", subcore_axis_name="s",
                                  num_cores=NC, num_subcores=NS)


def _wid():
    return lax.axis_index("c") * NS + lax.axis_index("s")


_SC_PARAMS = pltpu.CompilerParams(needs_layout_passes=False)


def _sc_gather_rows(table, idx, chunk):
    V, D = table.shape
    B = idx.shape[0]
    per_w = B // NW
    n_chunks = per_w // chunk
    n_outer = (n_chunks + 1) // 2

    @functools.partial(
        pl.kernel,
        compiler_params=_SC_PARAMS,
        out_type=jax.ShapeDtypeStruct((B, D), jnp.float32),
        mesh=_sc_mesh(),
        scratch_types=[
            pltpu.VMEM((per_w,), jnp.int32),
            pltpu.VMEM((chunk, D), jnp.float32),
            pltpu.VMEM((chunk, D), jnp.float32),
            pltpu.SemaphoreType.DMA,
            pltpu.SemaphoreType.DMA,
        ],
    )
    def k(table_hbm, idx_hbm, out_hbm, idx_v, rows0, rows1, sem0, sem1):
        base = _wid() * per_w
        pltpu.sync_copy(idx_hbm.at[pl.ds(pl.multiple_of(base, 8), per_w)],
                        idx_v)
        rows = (rows0, rows1)
        sems = (sem0, sem1)

        def start(i, b):
            pltpu.async_copy(
                table_hbm.at[idx_v.at[pl.ds(i * chunk, chunk)]],
                rows[b], sems[b])

        def finish(i, b):
            pltpu.make_async_copy(
                table_hbm.at[idx_v.at[pl.ds(0, chunk)]],
                rows[b], sems[b]).wait()
            off = pl.multiple_of(base + i * chunk, 8)
            pltpu.sync_copy(rows[b], out_hbm.at[pl.ds(off, chunk)])

        start(0, 0)

        def outer(o, carry):
            i0 = 2 * o

            @pl.when(i0 + 1 < n_chunks)
            def _():
                start(i0 + 1, 1)
            finish(i0, 0)

            @pl.when(i0 + 2 < n_chunks)
            def _():
                start(i0 + 2, 0)

            @pl.when(i0 + 1 < n_chunks)
            def _():
                finish(i0 + 1, 1)
            return carry

        lax.fori_loop(0, n_outer, outer, 0)

    return k(table, idx)


def _sc_scatter_graph(msg, dst, n_half, want_deg):
    FP = 64
    shift = n_half - 1
    n_units = T * 2 * n_half

    out_types = [jax.ShapeDtypeStruct((n_half, T, 2, NPHP * FP), jnp.float32)]
    if want_deg:
        out_types.append(jax.ShapeDtypeStruct((T * 2 * NPHP,), jnp.float32))

    @functools.partial(
        pl.kernel,
        compiler_params=_SC_PARAMS,
        out_type=tuple(out_types),
        mesh=_sc_mesh(),
        scratch_types=[
            pltpu.VMEM((CE,), jnp.int32),
            pltpu.VMEM((CE,), jnp.int32),
            pltpu.VMEM((CE, FP), jnp.float32),
            pltpu.VMEM((CE, FP), jnp.float32),
            pltpu.VMEM((NPHP * FP,), jnp.float32),
            pltpu.VMEM((NPHP,), jnp.float32),
            pltpu.SemaphoreType.DMA,
            pltpu.SemaphoreType.DMA,
        ],
    )
    def k(msg_hbm, dst_hbm, *refs):
        if want_deg:
            (agg_hbm, deg_hbm, dst_v0, dst_v1, msg_v0, msg_v1, acc, dacc,
             sem0, sem1) = refs
        else:
            (agg_hbm, dst_v0, dst_v1, msg_v0, msg_v1, acc, dacc,
             sem0, sem1) = refs
            deg_hbm = None
        dst_bufs = (dst_v0, dst_v1)
        msg_bufs = (msg_v0, msg_v1)
        sems = (sem0, sem1)
        w = _wid()
        n_my = ((n_units - 1 - w) >> 5) + 1
        iota = lax.iota(jnp.int32, LANES)
        ones = jnp.ones((LANES,), jnp.float32)
        zeros = jnp.zeros((LANES,), jnp.float32)

        def unit_body(ui, carry):
            u = w + ui * NW
            g = u >> (shift + 1)
            nh = (u >> shift) & 1
            fp = u & shift

            def zrow(r, c):
                for jj in range(FP // LANES):
                    acc[pl.ds((r * (FP // LANES) + jj) * LANES, LANES)] = zeros
                return c
            lax.fori_loop(0, NPHP, zrow, 0)
            if want_deg:
                def zdeg(r, c):
                    dacc[pl.ds(r * LANES, LANES)] = zeros
                    return c
                lax.fori_loop(0, NPHP // LANES, zdeg, 0)

            nhbase = g * NPG + nh * NPH

            def start(ci, p):
                off = pl.multiple_of(g * EPG + ci * CE, 8)
                pltpu.async_copy(dst_hbm.at[pl.ds(off, CE)], dst_bufs[p],
                                 sems[p])
                pltpu.async_copy(msg_hbm.at[fp, pl.ds(off, CE)], msg_bufs[p],
                                 sems[p])

            def wait(p):
                pltpu.make_async_copy(dst_hbm.at[pl.ds(0, CE)], dst_bufs[p],
                                      sems[p]).wait()
                pltpu.make_async_copy(msg_hbm.at[0, pl.ds(0, CE)], msg_bufs[p],
                                      sems[p]).wait()

            def process(dst_v, msg_v, c2):
                def blk_body(b, c3):
                    d16 = dst_v[pl.ds(b * LANES, LANES)] - nhbase
                    valid = (d16 >= 0) & (d16 < NPH)
                    d16c = jnp.where(valid, d16, NPH)
                    dflat = d16c * FP
                    if want_deg:
                        plsc.addupdate_scatter(dacc, [d16c], ones, mask=valid)
                    for l in range(LANES):
                        dl = dflat.at[jnp.full((LANES,), l, jnp.int32)].get(
                            mode="promise_in_bounds") + iota
                        for jj in range(FP // LANES):
                            v = msg_v[b * LANES + l, pl.ds(jj * LANES, LANES)]
                            plsc.addupdate_scatter(acc, [dl + jj * LANES], v)
                    return c3

                lax.fori_loop(0, NB, blk_body, 0)
                return c2

            start(0, 0)

            def chunk_pair(o, c2):
                for p in (0, 1):
                    ci = 2 * o + p

                    @pl.when(ci + 1 < NCH)
                    def _():
                        start(ci + 1, 1 - p)
                    wait(p)
                    process(dst_bufs[p], msg_bufs[p], c2)
                return c2

            lax.fori_loop(0, NCH // 2, chunk_pair, 0)

            pltpu.sync_copy(acc, agg_hbm.at[fp, g, nh])
            if want_deg:
                doff = pl.multiple_of((g * 2 + nh) * NPHP, 8)
                pltpu.sync_copy(dacc, deg_hbm.at[pl.ds(doff, NPHP)])
            return carry

        lax.fori_loop(0, n_my, unit_body, 0)

    outs = k(msg, dst)
    agg = outs[0].reshape(n_half, T, 2, NPHP, FP)
    return (agg, outs[1]) if want_deg else (agg,)


def _sc_type_mask(types):
    @functools.partial(
        pl.kernel,
        compiler_params=_SC_PARAMS,
        out_type=jax.ShapeDtypeStruct((NW, 2 * T, 128), jnp.float32),
        mesh=_sc_mesh(),
        scratch_types=[
            pltpu.VMEM((CE,), jnp.int32),
            pltpu.VMEM((2 * T, 128), jnp.float32),
        ],
    )
    def k(types_hbm, out_hbm, tv, macc):
        w = _wid()
        n_my = ((T - 1 - w) >> 5) + 1
        ones = jnp.ones((LANES,), jnp.float32)
        zeros = jnp.zeros((LANES,), jnp.float32)

        def zrow(r, c):
            for jj in range(128 // LANES):
                macc[r, pl.ds(jj * LANES, LANES)] = zeros
            return c
        lax.fori_loop(0, 2 * T, zrow, 0)

        def unit_body(ui, carry):
            g = w + ui * NW
            goff = g * N_REL

            def chunk_body(ci, c):
                off = pl.multiple_of(g * EPG + ci * CE, 8)
                pltpu.sync_copy(types_hbm.at[pl.ds(off, CE)], tv)

                def blk(b, c2):
                    t16 = tv[pl.ds(b * LANES, LANES)] + goff
                    plsc.addupdate_scatter(
                        macc, [t16 >> 7, t16 & 127], ones)
                    return c2

                lax.fori_loop(0, NB, blk, 0)
                return c

            lax.fori_loop(0, NCH, chunk_body, 0)
            return carry

        lax.fori_loop(0, n_my, unit_body, 0)
        pltpu.sync_copy(macc, out_hbm.at[w])

    return k(types)


def _tc_precompute(ent_embeds, ent_memory, rel_embeds, rel_memory,
                   Wn, bn, Wr, br, Wre1, Wre2):
    def body(ee, em, re, rm, wn, bn_, wr, br_, w1, w2,
             he_o, re0_o, re1_o, re2_o):
        he_o[...] = _DOT(ee[...], wn[0:2 * H, :]) + \
            _DOT(em[...], wn[2 * H:3 * H, :]) + bn_[...]
        re0 = _DOT(re[...], wr[0:2 * H, :]) + \
            _DOT(rm[...], wr[2 * H:3 * H, :]) + br_[...]
        re0_o[...] = re0
        re1 = jnp.maximum(_DOT(re0, w1[...]), 0.0)
        re1_o[...] = re1
        re2_o[...] = jnp.maximum(_DOT(re1, w2[...]), 0.0)

    n_ent = ent_embeds.shape[0]
    BN = 2000
    return pl.pallas_call(
        body,
        grid=(n_ent // BN,),
        in_specs=[
            pl.BlockSpec((BN, 2 * H), lambda i: (i, 0)),
            pl.BlockSpec((BN, H), lambda i: (i, 0)),
            pl.BlockSpec((N_REL, 2 * H), lambda i: (0, 0)),
            pl.BlockSpec((N_REL, H), lambda i: (0, 0)),
            pl.BlockSpec((3 * H, H), lambda i: (0, 0)),
            pl.BlockSpec((1, H), lambda i: (0, 0)),
            pl.BlockSpec((3 * H, H), lambda i: (0, 0)),
            pl.BlockSpec((1, H), lambda i: (0, 0)),
            pl.BlockSpec((H, OUT), lambda i: (0, 0)),
            pl.BlockSpec((OUT, H), lambda i: (0, 0)),
        ],
        out_specs=(
            pl.BlockSpec((BN, H), lambda i: (i, 0)),
            pl.BlockSpec((N_REL, H), lambda i: (0, 0)),
            pl.BlockSpec((N_REL, OUT), lambda i: (0, 0)),
            pl.BlockSpec((N_REL, H), lambda i: (0, 0)),
        ),
        out_shape=(
            jax.ShapeDtypeStruct((n_ent, H), jnp.float32),
            jax.ShapeDtypeStruct((N_REL, H), jnp.float32),
            jax.ShapeDtypeStruct((N_REL, OUT), jnp.float32),
            jax.ShapeDtypeStruct((N_REL, H), jnp.float32),
        ),
    )(ent_embeds, ent_memory, rel_embeds, rel_memory,
      Wn, bn.reshape(1, H), Wr, br.reshape(1, H), Wre1, Wre2)


def _tc_msg(hs, types_col, re_tab, text, Wm, Wt, d_use, n_half):
    BE = 3200
    FP = 64
    wm3 = jnp.moveaxis(Wm.reshape(d_use, n_half, FP), 1, 0)
    wt3 = jnp.moveaxis(Wt.reshape(SENT, n_half, FP), 1, 0)

    def body(hs_r, t_r, re_r, tx_r, wm_r, wt_r, out_r):
        oh = (lax.broadcasted_iota(jnp.int32, (BE, N_REL), 1) ==
              t_r[...]).astype(jnp.float32)
        es = _DOT(oh, re_r[...])
        prod = hs_r[...][:, :d_use] * es
        outs = [_DOT(prod, wm_r[...][f]) + _DOT(tx_r[...], wt_r[...][f])
                for f in range(n_half)]
        out_r[...] = jnp.stack(outs, axis=0)

    return pl.pallas_call(
        body,
        grid=(E // BE,),
        in_specs=[
            pl.BlockSpec((BE, H), lambda i: (i, 0)),
            pl.BlockSpec((BE, 1), lambda i: (i, 0)),
            pl.BlockSpec((N_REL, d_use), lambda i: (0, 0)),
            pl.BlockSpec((BE, SENT), lambda i: (i, 0)),
            pl.BlockSpec((n_half, d_use, FP), lambda i: (0, 0, 0)),
            pl.BlockSpec((n_half, SENT, FP), lambda i: (0, 0, 0)),
        ],
        out_specs=pl.BlockSpec((n_half, BE, FP), lambda i: (0, i, 0)),
        out_shape=jax.ShapeDtypeStruct((n_half, E, FP), jnp.float32),
    )(hs, types_col, re_tab, text, wm3, wt3)


def _assemble_agg(a, n_half):
    halves = []
    for f in range(n_half):
        rows = jnp.concatenate([a[f, 0, 0, :NPH, :], a[f, 0, 1, :NPH, :]],
                               axis=0)
        halves.append(rows)
    return jnp.concatenate(halves, axis=-1) if n_half > 1 else halves[0]


def _assemble_deg(dr):
    d = jnp.concatenate([dr[0, 0, :NPH], dr[0, 0, NPHP:NPHP + NPH]], axis=0)
    return jnp.maximum(d, 1.0)[:, None]


def _tc_update(agg5, deg3, h3d, Wl, d_in, d_out, n_half, pad_to=None):
    width = pad_to or d_out

    def body(agg_r, deg_r, h_r, wl_r, out_r):
        agg = _assemble_agg(agg_r[...], n_half)
        d = _assemble_deg(deg_r[...])
        h2 = jnp.maximum(agg / d + _DOT(h_r[...][0], wl_r[...]), 0.0)
        if width > d_out:
            h2 = jnp.concatenate(
                [h2, jnp.zeros((NPG, width - d_out), jnp.float32)], axis=-1)
        out_r[...] = h2[None]

    return pl.pallas_call(
        body,
        grid=(T,),
        in_specs=[
            pl.BlockSpec((n_half, 1, 2, NPHP, 64), lambda i: (0, i, 0, 0, 0)),
            pl.BlockSpec((1, 1, 2 * NPHP), lambda i: (i, 0, 0)),
            pl.BlockSpec((1, NPG, d_in), lambda i: (i, 0, 0)),
            pl.BlockSpec((d_in, d_out), lambda i: (0, 0)),
        ],
        out_specs=pl.BlockSpec((1, NPG, width), lambda i: (i, 0, 0)),
        out_shape=jax.ShapeDtypeStruct((T, NPG, width), jnp.float32),
    )(agg5, deg3, h3d, Wl)


def _tc_pool(agg5, deg3, h2_3d, Wl2, mask2d, re2):
    def body(agg_r, deg_r, h2_r, wl_r, mask_r, re2_r, g_o):
        i = pl.program_id(0)
        agg = _assemble_agg(agg_r[...], 2)
        d = _assemble_deg(deg_r[...])
        h3 = jnp.maximum(agg / d + _DOT(h2_r[...][0][:, :OUT], wl_r[...]),
                         0.0)
        gn = jnp.max(h3, axis=0, keepdims=True)
        m = jnp.sum(mask_r[:, pl.ds(i * N_REL, N_REL)], axis=0)
        ge = jnp.max(jnp.where(m[:, None] > 0.0, re2_r[...], -3.4e38),
                     axis=0, keepdims=True)
        g_o[...] = jnp.concatenate([gn, ge], axis=1)[None]

    return pl.pallas_call(
        body,
        grid=(T,),
        in_specs=[
            pl.BlockSpec((2, 1, 2, NPHP, 64), lambda i: (0, i, 0, 0, 0)),
            pl.BlockSpec((1, 1, 2 * NPHP), lambda i: (i, 0, 0)),
            pl.BlockSpec((1, NPG, H), lambda i: (i, 0, 0)),
            pl.BlockSpec((OUT, H), lambda i: (0, 0)),
            pl.BlockSpec((NW, MT), lambda i: (0, 0)),
            pl.BlockSpec((N_REL, H), lambda i: (0, 0)),
        ],
        out_specs=pl.BlockSpec((1, 1, 2 * H), lambda i: (i, 0, 0)),
        out_shape=jax.ShapeDtypeStruct((T, 1, 2 * H), jnp.float32),
    )(agg5, deg3, h2_3d, Wl2, mask2d, re2)


def _tc_final(g, time_idx_col, n_q):
    def body(g_r, ti_r, out_r):
        oh = (lax.broadcasted_iota(jnp.int32, (n_q, T), 1) ==
              ti_r[...]).astype(jnp.float32)
        out_r[...] = _DOT(oh, g_r[...])

    return pl.pallas_call(
        body,
        out_shape=jax.ShapeDtypeStruct((n_q, 2 * H), jnp.float32),
    )(g, time_idx_col)


def kernel(node_ids, edge_types, edge_index, node_graph_ids, edge_graph_ids,
           time_idx, ent_embeds, ent_memory, rel_embeds, rel_memory, text_emb,
           Wn, bn, Wr, br, Wm1, Wl1, Wre1, Wt1, Wm2, Wl2, Wre2, Wt2):
    src = edge_index[0].astype(jnp.int32)
    dst = edge_index[1].astype(jnp.int32)
    types = edge_types.astype(jnp.int32)
    text = text_emb.astype(jnp.float32)

    he, re0, re1, re2 = _tc_precompute(
        ent_embeds.astype(jnp.float32), ent_memory.astype(jnp.float32),
        rel_embeds.astype(jnp.float32), rel_memory.astype(jnp.float32),
        Wn, bn, Wr, br, Wre1, Wre2)

    NPAD = 50176
    nid_pad = jnp.pad(node_ids.astype(jnp.int32), (0, NPAD - N))
    h1 = _sc_gather_rows(he, nid_pad, 224)[:N]

    hs1 = _sc_gather_rows(h1, src, 200)
    mask_part = _sc_type_mask(types)
    types_col = types.reshape(E, 1)

    msg1 = _tc_msg(hs1, types_col, re0, text, Wm1, Wt1, H, 1)
    agg1, deg1d = _sc_scatter_graph(msg1, dst, 1, True)
    deg3 = deg1d.reshape(T, 1, 2 * NPHP)
    h2_3d = _tc_update(agg1, deg3, h1.reshape(T, NPG, H), Wl1, H, OUT, 1,
                       pad_to=H)
    h2 = h2_3d.reshape(N, H)

    hs2 = _sc_gather_rows(h2, src, 200)
    msg2 = _tc_msg(hs2, types_col, re1, text, Wm2, Wt2, OUT, 2)
    agg2 = _sc_scatter_graph(msg2, dst, 2, False)[0]

    g = _tc_pool(agg2, deg3, h2_3d, Wl2,
                 mask_part.reshape(NW, MT), re2).reshape(T, 2 * H)
    bq, sq = time_idx.shape
    ti_col = time_idx.astype(jnp.int32).reshape(bq * sq, 1)
    out = _tc_final(g, ti_col, bq * sq)
    return out.reshape(bq, sq, 2 * H)
